# initial kernel scaffold (unmeasured)
import jax
import jax.numpy as jnp
from jax import lax
from jax.experimental import pallas as pl
from jax.experimental.pallas import tpu as pltpu

N_DEV = 8


def kernel(x, w_mat):
    m, _ = x.shape
    _, n = w_mat.shape
    ch = m // N_DEV

    def body(x_ref, w_ref, out_ref, comm_ref, send_sems, recv_sems):
        my = lax.axis_index("i")
        left = lax.rem(my + N_DEV - 1, N_DEV)
        right = lax.rem(my + 1, N_DEV)

        barrier_sem = pltpu.get_barrier_semaphore()
        for nbr in (left, right):
            pl.semaphore_signal(
                barrier_sem, inc=1,
                device_id=(nbr,), device_id_type=pl.DeviceIdType.MESH,
            )
        pl.semaphore_wait(barrier_sem, 2)

        def partial(c):
            return jnp.dot(
                x_ref[pl.ds(c * ch, ch), :], w_ref[...],
                preferred_element_type=jnp.float32,
            )

        comm_ref[0, :, :] = partial(left).astype(jnp.bfloat16)

        for s in range(N_DEV - 1):
            send_slot = s % 2
            recv_slot = (s + 1) % 2
            rdma = pltpu.make_async_remote_copy(
                src_ref=comm_ref.at[send_slot],
                dst_ref=comm_ref.at[recv_slot],
                send_sem=send_sems.at[send_slot],
                recv_sem=recv_sems.at[recv_slot],
                device_id=(right,),
                device_id_type=pl.DeviceIdType.MESH,
            )
            rdma.start()
            c = lax.rem(my + 2 * N_DEV - s - 2, N_DEV)
            p = partial(c)
            rdma.wait()
            acc = comm_ref[recv_slot, :, :].astype(jnp.float32) + p
            if s < N_DEV - 2:
                comm_ref[recv_slot, :, :] = acc.astype(jnp.bfloat16)
            else:
                out_ref[...] = jax.nn.gelu(acc, approximate=True)

    return pl.pallas_call(
        body,
        out_shape=jax.ShapeDtypeStruct((ch, n), jnp.float32),
        in_specs=[
            pl.BlockSpec(memory_space=pltpu.VMEM),
            pl.BlockSpec(memory_space=pltpu.VMEM),
        ],
        out_specs=pl.BlockSpec(memory_space=pltpu.VMEM),
        scratch_shapes=[
            pltpu.VMEM((2, ch, n), jnp.bfloat16),
            pltpu.SemaphoreType.DMA((2,)),
            pltpu.SemaphoreType.DMA((2,)),
        ],
        compiler_params=pltpu.CompilerParams(collective_id=0),
    )(x, w_mat)


# baseline (device time: 720537 ns/iter reference)
import jax
import jax.numpy as jnp
from jax import lax
from jax.experimental import pallas as pl
from jax.experimental.pallas import tpu as pltpu

N_DEV = 8


def kernel(x, w_mat):
    m, _ = x.shape
    _, n = w_mat.shape
    ch = m // N_DEV

    x = x.astype(jnp.bfloat16)
    w_mat = w_mat.astype(jnp.bfloat16)

    def body(x_ref, w_ref, out_ref, xbuf, comm_ref, acc_ref,
             xsem, osem, send_sems, recv_sems):
        my = lax.axis_index("i")
        left = lax.rem(my + N_DEV - 1, N_DEV)
        right = lax.rem(my + 1, N_DEV)

        barrier_sem = pltpu.get_barrier_semaphore()
        for nbr in (left, right):
            pl.semaphore_signal(
                barrier_sem, inc=1,
                device_id=(nbr,), device_id_type=pl.DeviceIdType.MESH,
            )
        pl.semaphore_wait(barrier_sem, 2)

        def load_chunk(k):
            c = lax.rem(my + 2 * N_DEV - 1 - k, N_DEV)
            cp = pltpu.make_async_copy(
                x_ref.at[pl.ds(c * ch, ch), :], xbuf, xsem
            )
            cp.start()
            cp.wait()

        load_chunk(0)
        comm_ref[0, :, :] = jnp.dot(
            xbuf[...], w_ref[...], preferred_element_type=jnp.float32
        ).astype(jnp.bfloat16)

        def hop(s):
            send_slot = lax.rem(s, 2)
            recv_slot = lax.rem(s + 1, 2)
            rdma = pltpu.make_async_remote_copy(
                src_ref=comm_ref.at[send_slot],
                dst_ref=comm_ref.at[recv_slot],
                send_sem=send_sems.at[send_slot],
                recv_sem=recv_sems.at[recv_slot],
                device_id=(right,),
                device_id_type=pl.DeviceIdType.MESH,
            )
            rdma.start()
            load_chunk(s + 1)
            acc_ref[...] = jnp.dot(
                xbuf[...], w_ref[...], preferred_element_type=jnp.float32
            )
            rdma.wait()
            return recv_slot

        def middle_step(s, carry):
            recv_slot = hop(s)
            comm_ref[recv_slot, :, :] = (
                comm_ref[recv_slot, :, :].astype(jnp.float32) + acc_ref[...]
            ).astype(jnp.bfloat16)
            return carry

        lax.fori_loop(0, N_DEV - 2, middle_step, 0)

        recv_slot = hop(N_DEV - 2)
        acc_ref[...] = jax.nn.gelu(
            comm_ref[recv_slot, :, :].astype(jnp.float32) + acc_ref[...],
            approximate=True,
        )
        ocp = pltpu.make_async_copy(acc_ref, out_ref, osem)
        ocp.start()
        ocp.wait()

    return pl.pallas_call(
        body,
        out_shape=jax.ShapeDtypeStruct((ch, n), jnp.float32),
        in_specs=[
            pl.BlockSpec(memory_space=pl.ANY),
            pl.BlockSpec(memory_space=pltpu.MemorySpace.VMEM),
        ],
        out_specs=pl.BlockSpec(memory_space=pl.ANY),
        scratch_shapes=[
            pltpu.VMEM((ch, ch), jnp.bfloat16),
            pltpu.VMEM((2, ch, n), jnp.bfloat16),
            pltpu.VMEM((ch, n), jnp.float32),
            pltpu.SemaphoreType.DMA,
            pltpu.SemaphoreType.DMA,
            pltpu.SemaphoreType.DMA((2,)),
            pltpu.SemaphoreType.DMA((2,)),
        ],
        compiler_params=pltpu.CompilerParams(
            collective_id=0,
            vmem_limit_bytes=55 * 1024 * 1024,
        ),
    )(x, w_mat)


# device time: 409577 ns/iter; 1.7592x vs baseline; 1.7592x over previous
import jax
import jax.numpy as jnp
from jax import lax
from jax.experimental import pallas as pl
from jax.experimental.pallas import tpu as pltpu

N_DEV = 8


def kernel(x, w_mat):
    m, _ = x.shape
    _, n = w_mat.shape
    ch = m // N_DEV
    n2 = n // 2

    x = x.astype(jnp.bfloat16)
    w_mat = w_mat.astype(jnp.bfloat16)

    def body(x_ref, w_ref, out_ref, xbuf, cw_ref, ccw_ref, accl_ref,
             accr_ref, xsems, osems, cw_send, cw_recv, ccw_send, ccw_recv):
        my = lax.axis_index("i")
        left = lax.rem(my + N_DEV - 1, N_DEV)
        right = lax.rem(my + 1, N_DEV)

        barrier_sem = pltpu.get_barrier_semaphore()
        for nbr in (left, right):
            pl.semaphore_signal(
                barrier_sem, inc=1,
                device_id=(nbr,), device_id_type=pl.DeviceIdType.MESH,
            )
        pl.semaphore_wait(barrier_sem, 2)

        def compute_partials(k):
            c_cw = lax.rem(my + 2 * N_DEV - 1 - k, N_DEV)
            c_ccw = lax.rem(my + 1 + k, N_DEV)
            cp0 = pltpu.make_async_copy(
                x_ref.at[pl.ds(c_cw * ch, ch), :], xbuf.at[0], xsems.at[0]
            )
            cp1 = pltpu.make_async_copy(
                x_ref.at[pl.ds(c_ccw * ch, ch), :], xbuf.at[1], xsems.at[1]
            )
            cp0.start()
            cp1.start()
            cp0.wait()
            accl_ref[...] = jnp.dot(
                xbuf[0], w_ref[:, :n2], preferred_element_type=jnp.float32
            )
            cp1.wait()
            accr_ref[...] = jnp.dot(
                xbuf[1], w_ref[:, n2:], preferred_element_type=jnp.float32
            )

        compute_partials(0)
        cw_ref[0, :, :] = accl_ref[...].astype(jnp.bfloat16)
        ccw_ref[0, :, :] = accr_ref[...].astype(jnp.bfloat16)

        def hop(s):
            send_slot = lax.rem(s, 2)
            recv_slot = lax.rem(s + 1, 2)
            rd_cw = pltpu.make_async_remote_copy(
                src_ref=cw_ref.at[send_slot],
                dst_ref=cw_ref.at[recv_slot],
                send_sem=cw_send.at[send_slot],
                recv_sem=cw_recv.at[recv_slot],
                device_id=(right,),
                device_id_type=pl.DeviceIdType.MESH,
            )
            rd_ccw = pltpu.make_async_remote_copy(
                src_ref=ccw_ref.at[send_slot],
                dst_ref=ccw_ref.at[recv_slot],
                send_sem=ccw_send.at[send_slot],
                recv_sem=ccw_recv.at[recv_slot],
                device_id=(left,),
                device_id_type=pl.DeviceIdType.MESH,
            )
            rd_cw.start()
            rd_ccw.start()
            compute_partials(s + 1)
            rd_cw.wait()
            rd_ccw.wait()
            return recv_slot

        def middle_step(s, carry):
            recv_slot = hop(s)
            cw_ref[recv_slot, :, :] = (
                cw_ref[recv_slot, :, :].astype(jnp.float32) + accl_ref[...]
            ).astype(jnp.bfloat16)
            ccw_ref[recv_slot, :, :] = (
                ccw_ref[recv_slot, :, :].astype(jnp.float32) + accr_ref[...]
            ).astype(jnp.bfloat16)
            return carry

        lax.fori_loop(0, N_DEV - 2, middle_step, 0)

        recv_slot = hop(N_DEV - 2)
        accl_ref[...] = jax.nn.gelu(
            cw_ref[recv_slot, :, :].astype(jnp.float32) + accl_ref[...],
            approximate=True,
        )
        accr_ref[...] = jax.nn.gelu(
            ccw_ref[recv_slot, :, :].astype(jnp.float32) + accr_ref[...],
            approximate=True,
        )
        ocp0 = pltpu.make_async_copy(accl_ref, out_ref.at[:, :n2], osems.at[0])
        ocp1 = pltpu.make_async_copy(accr_ref, out_ref.at[:, n2:], osems.at[1])
        ocp0.start()
        ocp1.start()
        ocp0.wait()
        ocp1.wait()

    return pl.pallas_call(
        body,
        out_shape=jax.ShapeDtypeStruct((ch, n), jnp.float32),
        in_specs=[
            pl.BlockSpec(memory_space=pl.ANY),
            pl.BlockSpec(memory_space=pltpu.MemorySpace.VMEM),
        ],
        out_specs=pl.BlockSpec(memory_space=pl.ANY),
        scratch_shapes=[
            pltpu.VMEM((2, ch, ch), jnp.bfloat16),
            pltpu.VMEM((2, ch, n2), jnp.bfloat16),
            pltpu.VMEM((2, ch, n2), jnp.bfloat16),
            pltpu.VMEM((ch, n2), jnp.float32),
            pltpu.VMEM((ch, n2), jnp.float32),
            pltpu.SemaphoreType.DMA((2,)),
            pltpu.SemaphoreType.DMA((2,)),
            pltpu.SemaphoreType.DMA((2,)),
            pltpu.SemaphoreType.DMA((2,)),
            pltpu.SemaphoreType.DMA((2,)),
            pltpu.SemaphoreType.DMA((2,)),
        ],
        compiler_params=pltpu.CompilerParams(
            collective_id=0,
            vmem_limit_bytes=55 * 1024 * 1024,
        ),
    )(x, w_mat)


# device time: 398807 ns/iter; 1.8067x vs baseline; 1.0270x over previous
import jax
import jax.numpy as jnp
from jax import lax
from jax.experimental import pallas as pl
from jax.experimental.pallas import tpu as pltpu

N_DEV = 8


def kernel(x, w_mat):
    m, _ = x.shape
    _, n = w_mat.shape
    ch = m // N_DEV
    n2 = n // 2

    x = x.astype(jnp.bfloat16)
    w_mat = w_mat.astype(jnp.bfloat16)

    def body(x_ref, w_ref, out_ref, xbuf, cw_ref, ccw_ref, accl_ref,
             accr_ref, xsems, osems, cw_send, cw_recv, ccw_send, ccw_recv):
        my = lax.axis_index("i")
        left = lax.rem(my + N_DEV - 1, N_DEV)
        right = lax.rem(my + 1, N_DEV)

        barrier_sem = pltpu.get_barrier_semaphore()
        for nbr in (left, right):
            pl.semaphore_signal(
                barrier_sem, inc=1,
                device_id=(nbr,), device_id_type=pl.DeviceIdType.MESH,
            )
        pl.semaphore_wait(barrier_sem, 2)

        def load_x(k_chunk, slot):
            cp = pltpu.make_async_copy(
                x_ref.at[pl.ds(k_chunk * ch, ch), :],
                xbuf.at[slot], xsems.at[slot],
            )
            cp.start()
            return cp

        def partial_cw(k):
            load_x(lax.rem(my + 2 * N_DEV - 1 - k, N_DEV), 0).wait()
            accl_ref[...] = jnp.dot(
                xbuf[0], w_ref[:, :n2], preferred_element_type=jnp.float32
            )

        def partial_ccw(k):
            load_x(lax.rem(my + 1 + k, N_DEV), 1).wait()
            accr_ref[...] = jnp.dot(
                xbuf[1], w_ref[:, n2:], preferred_element_type=jnp.float32
            )

        def rdma_cw(s):
            return pltpu.make_async_remote_copy(
                src_ref=cw_ref.at[lax.rem(s, 2)],
                dst_ref=cw_ref.at[lax.rem(s + 1, 2)],
                send_sem=cw_send.at[lax.rem(s, 2)],
                recv_sem=cw_recv.at[lax.rem(s + 1, 2)],
                device_id=(right,),
                device_id_type=pl.DeviceIdType.MESH,
            )

        def rdma_ccw(s):
            return pltpu.make_async_remote_copy(
                src_ref=ccw_ref.at[lax.rem(s, 2)],
                dst_ref=ccw_ref.at[lax.rem(s + 1, 2)],
                send_sem=ccw_send.at[lax.rem(s, 2)],
                recv_sem=ccw_recv.at[lax.rem(s + 1, 2)],
                device_id=(left,),
                device_id_type=pl.DeviceIdType.MESH,
            )

        partial_cw(0)
        cw_ref[0, :, :] = accl_ref[...].astype(jnp.bfloat16)
        rdma_cw(0).start()
        partial_ccw(0)
        ccw_ref[0, :, :] = accr_ref[...].astype(jnp.bfloat16)
        rdma_ccw(0).start()

        def middle_step(s, carry):
            recv_slot = lax.rem(s + 1, 2)
            partial_cw(s + 1)
            rdma_cw(s).wait()
            cw_ref[recv_slot, :, :] = (
                cw_ref[recv_slot, :, :].astype(jnp.float32) + accl_ref[...]
            ).astype(jnp.bfloat16)
            rdma_cw(s + 1).start()
            partial_ccw(s + 1)
            rdma_ccw(s).wait()
            ccw_ref[recv_slot, :, :] = (
                ccw_ref[recv_slot, :, :].astype(jnp.float32) + accr_ref[...]
            ).astype(jnp.bfloat16)
            rdma_ccw(s + 1).start()
            return carry

        lax.fori_loop(0, N_DEV - 2, middle_step, 0)

        s_fin = N_DEV - 2
        recv_slot = (s_fin + 1) % 2
        partial_cw(s_fin + 1)
        rdma_cw(s_fin).wait()
        accl_ref[...] = jax.nn.gelu(
            cw_ref[recv_slot, :, :].astype(jnp.float32) + accl_ref[...],
            approximate=True,
        )
        ocp0 = pltpu.make_async_copy(accl_ref, out_ref.at[:, :n2], osems.at[0])
        ocp0.start()
        partial_ccw(s_fin + 1)
        rdma_ccw(s_fin).wait()
        accr_ref[...] = jax.nn.gelu(
            ccw_ref[recv_slot, :, :].astype(jnp.float32) + accr_ref[...],
            approximate=True,
        )
        ocp1 = pltpu.make_async_copy(accr_ref, out_ref.at[:, n2:], osems.at[1])
        ocp1.start()
        ocp0.wait()
        ocp1.wait()

    return pl.pallas_call(
        body,
        out_shape=jax.ShapeDtypeStruct((ch, n), jnp.float32),
        in_specs=[
            pl.BlockSpec(memory_space=pl.ANY),
            pl.BlockSpec(memory_space=pltpu.MemorySpace.VMEM),
        ],
        out_specs=pl.BlockSpec(memory_space=pl.ANY),
        scratch_shapes=[
            pltpu.VMEM((2, ch, ch), jnp.bfloat16),
            pltpu.VMEM((2, ch, n2), jnp.bfloat16),
            pltpu.VMEM((2, ch, n2), jnp.bfloat16),
            pltpu.VMEM((ch, n2), jnp.float32),
            pltpu.VMEM((ch, n2), jnp.float32),
            pltpu.SemaphoreType.DMA((2,)),
            pltpu.SemaphoreType.DMA((2,)),
            pltpu.SemaphoreType.DMA((2,)),
            pltpu.SemaphoreType.DMA((2,)),
            pltpu.SemaphoreType.DMA((2,)),
            pltpu.SemaphoreType.DMA((2,)),
        ],
        compiler_params=pltpu.CompilerParams(
            collective_id=0,
            vmem_limit_bytes=55 * 1024 * 1024,
        ),
    )(x, w_mat)


# device time: 384894 ns/iter; 1.8720x vs baseline; 1.0361x over previous
import jax
import jax.numpy as jnp
from jax import lax
from jax.experimental import pallas as pl
from jax.experimental.pallas import tpu as pltpu

N_DEV = 8


def kernel(x, w_mat):
    m, _ = x.shape
    _, n = w_mat.shape
    ch = m // N_DEV
    n2 = n // 2

    x = x.astype(jnp.bfloat16)
    w_mat = w_mat.astype(jnp.bfloat16)

    def body(x_ref, w_ref, out_ref, xbuf, cw_ref, ccw_ref, accl_ref,
             accr_ref, xsems, osems, cw_send, cw_recv, ccw_send, ccw_recv):
        my = lax.axis_index("i")
        left = lax.rem(my + N_DEV - 1, N_DEV)
        right = lax.rem(my + 1, N_DEV)

        barrier_sem = pltpu.get_barrier_semaphore()
        for nbr in (left, right):
            pl.semaphore_signal(
                barrier_sem, inc=1,
                device_id=(nbr,), device_id_type=pl.DeviceIdType.MESH,
            )
        pl.semaphore_wait(barrier_sem, 2)

        def load_x(k_chunk, slot):
            cp = pltpu.make_async_copy(
                x_ref.at[pl.ds(k_chunk * ch, ch), :],
                xbuf.at[slot], xsems.at[slot],
            )
            cp.start()
            return cp

        def partial_cw(k):
            load_x(lax.rem(my + 2 * N_DEV - 1 - k, N_DEV), 0).wait()
            accl_ref[...] = jnp.dot(
                xbuf[0], w_ref[:, :n2], preferred_element_type=jnp.float32
            )

        def partial_ccw(k):
            load_x(lax.rem(my + 1 + k, N_DEV), 1).wait()
            accr_ref[...] = jnp.dot(
                xbuf[1], w_ref[:, n2:], preferred_element_type=jnp.float32
            )

        sch = n2 // 2

        def rdma_cw(s, sub):
            return pltpu.make_async_remote_copy(
                src_ref=cw_ref.at[lax.rem(s, 2), :, pl.ds(sub * sch, sch)],
                dst_ref=cw_ref.at[lax.rem(s + 1, 2), :, pl.ds(sub * sch, sch)],
                send_sem=cw_send.at[lax.rem(s, 2), sub],
                recv_sem=cw_recv.at[lax.rem(s + 1, 2), sub],
                device_id=(right,),
                device_id_type=pl.DeviceIdType.MESH,
            )

        def rdma_ccw(s, sub):
            return pltpu.make_async_remote_copy(
                src_ref=ccw_ref.at[lax.rem(s, 2), :, pl.ds(sub * sch, sch)],
                dst_ref=ccw_ref.at[lax.rem(s + 1, 2), :, pl.ds(sub * sch, sch)],
                send_sem=ccw_send.at[lax.rem(s, 2), sub],
                recv_sem=ccw_recv.at[lax.rem(s + 1, 2), sub],
                device_id=(left,),
                device_id_type=pl.DeviceIdType.MESH,
            )

        partial_cw(0)
        cw_ref[0, :, :] = accl_ref[...].astype(jnp.bfloat16)
        rdma_cw(0, 0).start()
        rdma_cw(0, 1).start()
        partial_ccw(0)
        ccw_ref[0, :, :] = accr_ref[...].astype(jnp.bfloat16)
        rdma_ccw(0, 0).start()
        rdma_ccw(0, 1).start()

        def middle_step(s, carry):
            recv_slot = lax.rem(s + 1, 2)
            partial_cw(s + 1)
            partial_ccw(s + 1)
            for sub in (0, 1):
                cs = pl.ds(sub * sch, sch)
                rdma_cw(s, sub).wait()
                cw_ref[recv_slot, :, cs] = (
                    cw_ref[recv_slot, :, cs].astype(jnp.float32)
                    + accl_ref[:, cs]
                ).astype(jnp.bfloat16)
                rdma_cw(s + 1, sub).start()
                rdma_ccw(s, sub).wait()
                ccw_ref[recv_slot, :, cs] = (
                    ccw_ref[recv_slot, :, cs].astype(jnp.float32)
                    + accr_ref[:, cs]
                ).astype(jnp.bfloat16)
                rdma_ccw(s + 1, sub).start()
            return carry

        lax.fori_loop(0, N_DEV - 2, middle_step, 0)

        s_fin = N_DEV - 2
        recv_slot = (s_fin + 1) % 2
        partial_cw(s_fin + 1)
        rdma_cw(s_fin, 0).wait()
        rdma_cw(s_fin, 1).wait()
        accl_ref[...] = jax.nn.gelu(
            cw_ref[recv_slot, :, :].astype(jnp.float32) + accl_ref[...],
            approximate=True,
        )
        ocp0 = pltpu.make_async_copy(accl_ref, out_ref.at[:, :n2], osems.at[0])
        ocp0.start()
        partial_ccw(s_fin + 1)
        rdma_ccw(s_fin, 0).wait()
        rdma_ccw(s_fin, 1).wait()
        accr_ref[...] = jax.nn.gelu(
            ccw_ref[recv_slot, :, :].astype(jnp.float32) + accr_ref[...],
            approximate=True,
        )
        ocp1 = pltpu.make_async_copy(accr_ref, out_ref.at[:, n2:], osems.at[1])
        ocp1.start()
        ocp0.wait()
        ocp1.wait()

    return pl.pallas_call(
        body,
        out_shape=jax.ShapeDtypeStruct((ch, n), jnp.float32),
        in_specs=[
            pl.BlockSpec(memory_space=pl.ANY),
            pl.BlockSpec(memory_space=pltpu.MemorySpace.VMEM),
        ],
        out_specs=pl.BlockSpec(memory_space=pl.ANY),
        scratch_shapes=[
            pltpu.VMEM((2, ch, ch), jnp.bfloat16),
            pltpu.VMEM((2, ch, n2), jnp.bfloat16),
            pltpu.VMEM((2, ch, n2), jnp.bfloat16),
            pltpu.VMEM((ch, n2), jnp.float32),
            pltpu.VMEM((ch, n2), jnp.float32),
            pltpu.SemaphoreType.DMA((2,)),
            pltpu.SemaphoreType.DMA((2,)),
            pltpu.SemaphoreType.DMA((2, 2)),
            pltpu.SemaphoreType.DMA((2, 2)),
            pltpu.SemaphoreType.DMA((2, 2)),
            pltpu.SemaphoreType.DMA((2, 2)),
        ],
        compiler_params=pltpu.CompilerParams(
            collective_id=0,
            vmem_limit_bytes=55 * 1024 * 1024,
        ),
    )(x, w_mat)


# device time: 377856 ns/iter; 1.9069x vs baseline; 1.0186x over previous
import jax
import jax.numpy as jnp
from jax import lax
from jax.experimental import pallas as pl
from jax.experimental.pallas import tpu as pltpu

N_DEV = 8


def kernel(x, w_mat):
    m, _ = x.shape
    _, n = w_mat.shape
    ch = m // N_DEV
    n2 = n // 2

    x = x.astype(jnp.bfloat16)
    w_mat = w_mat.astype(jnp.bfloat16)

    def body(x_ref, w_ref, out_ref, xbuf, cw_ref, ccw_ref, accl_ref,
             accr_ref, xsems, osems, cw_send, cw_recv, ccw_send, ccw_recv):
        my = lax.axis_index("i")
        left = lax.rem(my + N_DEV - 1, N_DEV)
        right = lax.rem(my + 1, N_DEV)

        barrier_sem = pltpu.get_barrier_semaphore()
        for nbr in (left, right):
            pl.semaphore_signal(
                barrier_sem, inc=1,
                device_id=(nbr,), device_id_type=pl.DeviceIdType.MESH,
            )
        pl.semaphore_wait(barrier_sem, 2)

        def load_x(k_chunk, slot):
            cp = pltpu.make_async_copy(
                x_ref.at[pl.ds(k_chunk * ch, ch), :],
                xbuf.at[slot], xsems.at[slot],
            )
            cp.start()
            return cp

        def partial_cw(k):
            load_x(lax.rem(my + 2 * N_DEV - 1 - k, N_DEV), 0).wait()
            accl_ref[...] = jnp.dot(
                xbuf[0], w_ref[:, :n2], preferred_element_type=jnp.float32
            )

        def partial_ccw(k):
            load_x(lax.rem(my + 1 + k, N_DEV), 1).wait()
            accr_ref[...] = jnp.dot(
                xbuf[1], w_ref[:, n2:], preferred_element_type=jnp.float32
            )

        sch = n2 // 2

        def rdma_cw(s, sub):
            return pltpu.make_async_remote_copy(
                src_ref=cw_ref.at[lax.rem(s, 2), :, pl.ds(sub * sch, sch)],
                dst_ref=cw_ref.at[lax.rem(s + 1, 2), :, pl.ds(sub * sch, sch)],
                send_sem=cw_send.at[lax.rem(s, 2), sub],
                recv_sem=cw_recv.at[lax.rem(s + 1, 2), sub],
                device_id=(right,),
                device_id_type=pl.DeviceIdType.MESH,
            )

        def rdma_ccw(s, sub):
            return pltpu.make_async_remote_copy(
                src_ref=ccw_ref.at[lax.rem(s, 2), :, pl.ds(sub * sch, sch)],
                dst_ref=ccw_ref.at[lax.rem(s + 1, 2), :, pl.ds(sub * sch, sch)],
                send_sem=ccw_send.at[lax.rem(s, 2), sub],
                recv_sem=ccw_recv.at[lax.rem(s + 1, 2), sub],
                device_id=(left,),
                device_id_type=pl.DeviceIdType.MESH,
            )

        load_x(lax.rem(my + N_DEV - 1, N_DEV), 0).wait()
        for sub in (0, 1):
            cs = pl.ds(sub * sch, sch)
            accl_ref[:, cs] = jnp.dot(
                xbuf[0], w_ref[:, sub * sch:(sub + 1) * sch],
                preferred_element_type=jnp.float32,
            )
            cw_ref[0, :, cs] = accl_ref[:, cs].astype(jnp.bfloat16)
            rdma_cw(0, sub).start()
        load_x(lax.rem(my + 1, N_DEV), 1).wait()
        for sub in (0, 1):
            cs = pl.ds(sub * sch, sch)
            accr_ref[:, cs] = jnp.dot(
                xbuf[1], w_ref[:, n2 + sub * sch:n2 + (sub + 1) * sch],
                preferred_element_type=jnp.float32,
            )
            ccw_ref[0, :, cs] = accr_ref[:, cs].astype(jnp.bfloat16)
            rdma_ccw(0, sub).start()

        def middle_step(s, carry):
            recv_slot = lax.rem(s + 1, 2)

            def handle(rdma_fn, ref, acc, sub):
                cs = pl.ds(sub * sch, sch)
                rdma_fn(s, sub).wait()
                ref[recv_slot, :, cs] = (
                    ref[recv_slot, :, cs].astype(jnp.float32) + acc[:, cs]
                ).astype(jnp.bfloat16)
                rdma_fn(s + 1, sub).start()

            partial_cw(s + 1)
            handle(rdma_cw, cw_ref, accl_ref, 0)
            partial_ccw(s + 1)
            handle(rdma_ccw, ccw_ref, accr_ref, 0)
            handle(rdma_cw, cw_ref, accl_ref, 1)
            handle(rdma_ccw, ccw_ref, accr_ref, 1)
            return carry

        lax.fori_loop(0, N_DEV - 2, middle_step, 0)

        s_fin = N_DEV - 2
        recv_slot = (s_fin + 1) % 2
        ocps = []

        def finish(rdma_fn, ref, acc, col0, sub):
            cs = pl.ds(sub * sch, sch)
            rdma_fn(s_fin, sub).wait()
            acc[:, cs] = jax.nn.gelu(
                ref[recv_slot, :, cs].astype(jnp.float32) + acc[:, cs],
                approximate=True,
            )
            ocp = pltpu.make_async_copy(
                acc.at[:, cs],
                out_ref.at[:, pl.ds(col0 + sub * sch, sch)],
                osems.at[len(ocps)],
            )
            ocp.start()
            ocps.append(ocp)

        partial_cw(s_fin + 1)
        finish(rdma_cw, cw_ref, accl_ref, 0, 0)
        partial_ccw(s_fin + 1)
        finish(rdma_ccw, ccw_ref, accr_ref, n2, 0)
        finish(rdma_cw, cw_ref, accl_ref, 0, 1)
        finish(rdma_ccw, ccw_ref, accr_ref, n2, 1)
        for ocp in ocps:
            ocp.wait()

    return pl.pallas_call(
        body,
        out_shape=jax.ShapeDtypeStruct((ch, n), jnp.float32),
        in_specs=[
            pl.BlockSpec(memory_space=pl.ANY),
            pl.BlockSpec(memory_space=pltpu.MemorySpace.VMEM),
        ],
        out_specs=pl.BlockSpec(memory_space=pl.ANY),
        scratch_shapes=[
            pltpu.VMEM((2, ch, ch), jnp.bfloat16),
            pltpu.VMEM((2, ch, n2), jnp.bfloat16),
            pltpu.VMEM((2, ch, n2), jnp.bfloat16),
            pltpu.VMEM((ch, n2), jnp.float32),
            pltpu.VMEM((ch, n2), jnp.float32),
            pltpu.SemaphoreType.DMA((2,)),
            pltpu.SemaphoreType.DMA((4,)),
            pltpu.SemaphoreType.DMA((2, 2)),
            pltpu.SemaphoreType.DMA((2, 2)),
            pltpu.SemaphoreType.DMA((2, 2)),
            pltpu.SemaphoreType.DMA((2, 2)),
        ],
        compiler_params=pltpu.CompilerParams(
            collective_id=0,
            vmem_limit_bytes=55 * 1024 * 1024,
        ),
    )(x, w_mat)


# device time: 376694 ns/iter; 1.9128x vs baseline; 1.0031x over previous
import jax
import jax.numpy as jnp
from jax import lax
from jax.experimental import pallas as pl
from jax.experimental.pallas import tpu as pltpu

N_DEV = 8


def kernel(x, w_mat):
    m, _ = x.shape
    _, n = w_mat.shape
    ch = m // N_DEV
    n2 = n // 2

    x = x.astype(jnp.bfloat16)
    w_mat = w_mat.astype(jnp.bfloat16)

    def body(x_ref, w_ref, out_ref, xbuf, cw_ref, ccw_ref, accl_ref,
             accr_ref, xsems, osems, cw_send, cw_recv, ccw_send, ccw_recv):
        my = lax.axis_index("i")
        left = lax.rem(my + N_DEV - 1, N_DEV)
        right = lax.rem(my + 1, N_DEV)

        barrier_sem = pltpu.get_barrier_semaphore()
        for nbr in (left, right):
            pl.semaphore_signal(
                barrier_sem, inc=1,
                device_id=(nbr,), device_id_type=pl.DeviceIdType.MESH,
            )

        def load_x(k_chunk, slot):
            cp = pltpu.make_async_copy(
                x_ref.at[pl.ds(k_chunk * ch, ch), :],
                xbuf.at[slot], xsems.at[slot],
            )
            cp.start()
            return cp

        def partial_cw(k):
            load_x(lax.rem(my + 2 * N_DEV - 1 - k, N_DEV), 0).wait()
            accl_ref[...] = jnp.dot(
                xbuf[0], w_ref[:, :n2], preferred_element_type=jnp.float32
            )

        def partial_ccw(k):
            load_x(lax.rem(my + 1 + k, N_DEV), 1).wait()
            accr_ref[...] = jnp.dot(
                xbuf[1], w_ref[:, n2:], preferred_element_type=jnp.float32
            )

        sch = n2 // 2

        def rdma_cw(s, sub):
            return pltpu.make_async_remote_copy(
                src_ref=cw_ref.at[lax.rem(s, 2), :, pl.ds(sub * sch, sch)],
                dst_ref=cw_ref.at[lax.rem(s + 1, 2), :, pl.ds(sub * sch, sch)],
                send_sem=cw_send.at[lax.rem(s, 2), sub],
                recv_sem=cw_recv.at[lax.rem(s + 1, 2), sub],
                device_id=(right,),
                device_id_type=pl.DeviceIdType.MESH,
            )

        def rdma_ccw(s, sub):
            return pltpu.make_async_remote_copy(
                src_ref=ccw_ref.at[lax.rem(s, 2), :, pl.ds(sub * sch, sch)],
                dst_ref=ccw_ref.at[lax.rem(s + 1, 2), :, pl.ds(sub * sch, sch)],
                send_sem=ccw_send.at[lax.rem(s, 2), sub],
                recv_sem=ccw_recv.at[lax.rem(s + 1, 2), sub],
                device_id=(left,),
                device_id_type=pl.DeviceIdType.MESH,
            )

        load_x(lax.rem(my + N_DEV - 1, N_DEV), 0).wait()
        for sub in (0, 1):
            cs = pl.ds(sub * sch, sch)
            accl_ref[:, cs] = jnp.dot(
                xbuf[0], w_ref[:, sub * sch:(sub + 1) * sch],
                preferred_element_type=jnp.float32,
            )
            cw_ref[0, :, cs] = accl_ref[:, cs].astype(jnp.bfloat16)
            if sub == 0:
                pl.semaphore_wait(barrier_sem, 2)
            rdma_cw(0, sub).start()
        load_x(lax.rem(my + 1, N_DEV), 1).wait()
        for sub in (0, 1):
            cs = pl.ds(sub * sch, sch)
            accr_ref[:, cs] = jnp.dot(
                xbuf[1], w_ref[:, n2 + sub * sch:n2 + (sub + 1) * sch],
                preferred_element_type=jnp.float32,
            )
            ccw_ref[0, :, cs] = accr_ref[:, cs].astype(jnp.bfloat16)
            rdma_ccw(0, sub).start()

        def middle_step(s, carry):
            recv_slot = lax.rem(s + 1, 2)

            def handle(rdma_fn, ref, acc, sub):
                cs = pl.ds(sub * sch, sch)
                rdma_fn(s, sub).wait()
                ref[recv_slot, :, cs] = (
                    ref[recv_slot, :, cs].astype(jnp.float32) + acc[:, cs]
                ).astype(jnp.bfloat16)
                rdma_fn(s + 1, sub).start()

            partial_cw(s + 1)
            handle(rdma_cw, cw_ref, accl_ref, 0)
            partial_ccw(s + 1)
            handle(rdma_ccw, ccw_ref, accr_ref, 0)
            handle(rdma_cw, cw_ref, accl_ref, 1)
            handle(rdma_ccw, ccw_ref, accr_ref, 1)
            return carry

        lax.fori_loop(0, N_DEV - 2, middle_step, 0)

        s_fin = N_DEV - 2
        recv_slot = (s_fin + 1) % 2
        ocps = []

        def finish(rdma_fn, ref, acc, col0, sub):
            cs = pl.ds(sub * sch, sch)
            rdma_fn(s_fin, sub).wait()
            acc[:, cs] = jax.nn.gelu(
                ref[recv_slot, :, cs].astype(jnp.float32) + acc[:, cs],
                approximate=True,
            )
            ocp = pltpu.make_async_copy(
                acc.at[:, cs],
                out_ref.at[:, pl.ds(col0 + sub * sch, sch)],
                osems.at[len(ocps)],
            )
            ocp.start()
            ocps.append(ocp)

        partial_cw(s_fin + 1)
        finish(rdma_cw, cw_ref, accl_ref, 0, 0)
        partial_ccw(s_fin + 1)
        finish(rdma_ccw, ccw_ref, accr_ref, n2, 0)
        finish(rdma_cw, cw_ref, accl_ref, 0, 1)
        finish(rdma_ccw, ccw_ref, accr_ref, n2, 1)
        for ocp in ocps:
            ocp.wait()

    return pl.pallas_call(
        body,
        out_shape=jax.ShapeDtypeStruct((ch, n), jnp.float32),
        in_specs=[
            pl.BlockSpec(memory_space=pl.ANY),
            pl.BlockSpec(memory_space=pltpu.MemorySpace.VMEM),
        ],
        out_specs=pl.BlockSpec(memory_space=pl.ANY),
        scratch_shapes=[
            pltpu.VMEM((2, ch, ch), jnp.bfloat16),
            pltpu.VMEM((2, ch, n2), jnp.bfloat16),
            pltpu.VMEM((2, ch, n2), jnp.bfloat16),
            pltpu.VMEM((ch, n2), jnp.float32),
            pltpu.VMEM((ch, n2), jnp.float32),
            pltpu.SemaphoreType.DMA((2,)),
            pltpu.SemaphoreType.DMA((4,)),
            pltpu.SemaphoreType.DMA((2, 2)),
            pltpu.SemaphoreType.DMA((2, 2)),
            pltpu.SemaphoreType.DMA((2, 2)),
            pltpu.SemaphoreType.DMA((2, 2)),
        ],
        compiler_params=pltpu.CompilerParams(
            collective_id=0,
            vmem_limit_bytes=55 * 1024 * 1024,
        ),
    )(x, w_mat)
